# initial kernel scaffold (unmeasured)
import jax
import jax.numpy as jnp
from jax import lax
from jax.experimental import pallas as pl
from jax.experimental.pallas import tpu as pltpu

B = 32
NB = 256
BS = 32
H = 16
D = 128
ZDIM = 4


def kernel(Q, K, V, bt, lens):
    n_local_pages = K.shape[0]
    my_z = lax.axis_index("z")

    pos = jnp.arange(NB, dtype=jnp.int32)[None, :]
    owned = (bt // n_local_pages == my_z) & (pos < lens[:, None])
    order = jnp.argsort(jnp.logical_not(owned), axis=1)
    cpt = jnp.take_along_axis(bt, order, axis=1) - my_z * n_local_pages
    cpt = jnp.clip(cpt, 0, n_local_pages - 1).astype(jnp.int32)
    counts = owned.sum(axis=1).astype(jnp.int32)

    scale = D ** -0.5

    def body(q_ref, k_ref, v_ref, cpt_ref, cnt_ref, out_ref,
             comm_ref, kbuf, vbuf, kv_sems, send_sems, recv_sems):
        my_x = lax.axis_index("x")
        my_y = lax.axis_index("y")
        mz = lax.axis_index("z")

        def start_dma(i, t, slot):
            p = cpt_ref[i, t]
            pltpu.make_async_copy(k_ref.at[p], kbuf.at[slot],
                                  kv_sems.at[0, slot]).start()
            pltpu.make_async_copy(v_ref.at[p], vbuf.at[slot],
                                  kv_sems.at[1, slot]).start()

        def wait_dma(slot):
            pltpu.make_async_copy(k_ref.at[0], kbuf.at[slot],
                                  kv_sems.at[0, slot]).wait()
            pltpu.make_async_copy(v_ref.at[0], vbuf.at[slot],
                                  kv_sems.at[1, slot]).wait()

        def batch_body(i, carry):
            T = cnt_ref[i]
            q = q_ref[i, 0]

            @pl.when(T > 0)
            def _():
                start_dma(i, 0, 0)

            def page_body(t, mla):
                m, l, acc = mla
                slot = lax.rem(t, 2)

                @pl.when(t + 1 < T)
                def _():
                    start_dma(i, t + 1, lax.rem(t + 1, 2))

                wait_dma(slot)
                kp = kbuf[slot]
                vp = vbuf[slot]
                s = jnp.sum(kp * q[None, :, :], axis=-1) * scale
                m_new = jnp.maximum(m, jnp.max(s, axis=0, keepdims=True))
                alpha = jnp.exp(m - m_new)
                p = jnp.exp(s - m_new)
                l_new = l * alpha + jnp.sum(p, axis=0, keepdims=True)
                acc_new = acc * alpha.reshape(H, 1) + \
                    jnp.sum(p[:, :, None] * vp, axis=0)
                return m_new, l_new, acc_new

            init = (jnp.full((1, H), -1e30, jnp.float32),
                    jnp.zeros((1, H), jnp.float32),
                    jnp.zeros((H, D), jnp.float32))
            m, l, acc = lax.fori_loop(0, T, page_body, init)

            comm_ref[0, 0, i] = acc
            comm_ref[0, 1, i] = jnp.broadcast_to(m.reshape(H, 1), (H, D))
            comm_ref[0, 2, i] = jnp.broadcast_to(l.reshape(H, 1), (H, D))
            return carry

        lax.fori_loop(0, B, batch_body, 0)

        bsem = pltpu.get_barrier_semaphore()
        for dz in (1, 2, 3):
            pl.semaphore_signal(
                bsem, inc=1,
                device_id=(my_x, my_y, lax.rem(mz + dz, ZDIM)),
                device_id_type=pl.DeviceIdType.MESH,
            )
        pl.semaphore_wait(bsem, 3)

        sends = []
        for dz in (1, 2, 3):
            rdma = pltpu.make_async_remote_copy(
                src_ref=comm_ref.at[0],
                dst_ref=comm_ref.at[ZDIM - dz],
                send_sem=send_sems.at[dz],
                recv_sem=recv_sems.at[ZDIM - dz],
                device_id=(my_x, my_y, lax.rem(mz + dz, ZDIM)),
                device_id_type=pl.DeviceIdType.MESH,
            )
            rdma.start()
            sends.append(rdma)

        for s in (1, 2, 3):
            recv = pltpu.make_async_remote_copy(
                src_ref=comm_ref.at[s],
                dst_ref=comm_ref.at[s],
                send_sem=send_sems.at[0],
                recv_sem=recv_sems.at[s],
                device_id=(my_x, my_y, mz),
                device_id_type=pl.DeviceIdType.MESH,
            )
            recv.wait_recv()
        for rdma in sends:
            rdma.wait_send()

        mall = comm_ref[:, 1]
        mmax = jnp.max(mall, axis=0)
        w = jnp.exp(mall - mmax[None])
        o = jnp.sum(comm_ref[:, 0] * w, axis=0)
        lsum = jnp.sum(comm_ref[:, 2] * w, axis=0)
        out_ref[:, 0] = o / lsum

    return pl.pallas_call(
        body,
        out_shape=jax.ShapeDtypeStruct((B, 1, H, D), jnp.float32),
        in_specs=[
            pl.BlockSpec(memory_space=pltpu.VMEM),
            pl.BlockSpec(memory_space=pltpu.ANY),
            pl.BlockSpec(memory_space=pltpu.ANY),
            pl.BlockSpec(memory_space=pltpu.SMEM),
            pl.BlockSpec(memory_space=pltpu.SMEM),
        ],
        out_specs=pl.BlockSpec(memory_space=pltpu.VMEM),
        scratch_shapes=[
            pltpu.VMEM((ZDIM, 3, B, H, D), jnp.float32),
            pltpu.VMEM((2, BS, H, D), jnp.float32),
            pltpu.VMEM((2, BS, H, D), jnp.float32),
            pltpu.SemaphoreType.DMA((2, 2)),
            pltpu.SemaphoreType.DMA((4,)),
            pltpu.SemaphoreType.DMA((4,)),
        ],
        compiler_params=pltpu.CompilerParams(collective_id=0),
    )(Q, K, V, cpt, counts)


# baseline (device time: 899753 ns/iter reference)
import jax
import jax.numpy as jnp
from jax import lax
from jax.experimental import pallas as pl
from jax.experimental.pallas import tpu as pltpu

B = 32
NB = 256
BS = 32
H = 16
D = 128
ZDIM = 4


def kernel(Q, K, V, bt, lens):
    n_local_pages = K.shape[0]
    my_z = lax.axis_index("z")

    pos = jnp.arange(NB, dtype=jnp.int32)[None, :]
    owned = (bt // n_local_pages == my_z) & (pos < lens[:, None])
    order = jnp.argsort(jnp.logical_not(owned), axis=1)
    cpt = jnp.take_along_axis(bt, order, axis=1) - my_z * n_local_pages
    cpt = jnp.clip(cpt, 0, n_local_pages - 1).astype(jnp.int32)
    counts = owned.sum(axis=1).astype(jnp.int32)

    scale = D ** -0.5

    def body(q_ref, k_ref, v_ref, cpt_ref, cnt_ref, out_ref,
             comm_ref, kbuf, vbuf, kv_sems, send_sems, recv_sems):
        my_x = lax.axis_index("x")
        my_y = lax.axis_index("y")
        mz = lax.axis_index("z")

        def start_dma(i, t, slot):
            p = cpt_ref[i, t]
            pltpu.make_async_copy(k_ref.at[p], kbuf.at[slot],
                                  kv_sems.at[0, slot]).start()
            pltpu.make_async_copy(v_ref.at[p], vbuf.at[slot],
                                  kv_sems.at[1, slot]).start()

        def wait_dma(slot):
            pltpu.make_async_copy(k_ref.at[0], kbuf.at[slot],
                                  kv_sems.at[0, slot]).wait()
            pltpu.make_async_copy(v_ref.at[0], vbuf.at[slot],
                                  kv_sems.at[1, slot]).wait()

        def batch_body(i, carry):
            T = cnt_ref[i]
            q = q_ref[i, 0]

            @pl.when(T > 0)
            def _():
                start_dma(i, 0, 0)

            def page_body(t, mla):
                m, l, acc = mla
                slot = lax.rem(t, 2)

                @pl.when(t + 1 < T)
                def _():
                    start_dma(i, t + 1, lax.rem(t + 1, 2))

                wait_dma(slot)
                kp = kbuf[slot]
                vp = vbuf[slot]
                s = jnp.sum(kp * q[None, :, :], axis=-1) * scale
                m_new = jnp.maximum(m, jnp.max(s, axis=0, keepdims=True))
                alpha = jnp.exp(m - m_new)
                p = jnp.exp(s - m_new)
                l_new = l * alpha + jnp.sum(p, axis=0, keepdims=True)
                acc_new = acc * alpha.reshape(H, 1) + \
                    jnp.sum(p[:, :, None] * vp, axis=0)
                return m_new, l_new, acc_new

            init = (jnp.full((1, H), -1e30, jnp.float32),
                    jnp.zeros((1, H), jnp.float32),
                    jnp.zeros((H, D), jnp.float32))
            m, l, acc = lax.fori_loop(0, T, page_body, init)

            comm_ref[0, 0, i] = acc
            comm_ref[0, 1, i] = jnp.broadcast_to(m.reshape(H, 1), (H, D))
            comm_ref[0, 2, i] = jnp.broadcast_to(l.reshape(H, 1), (H, D))
            return carry

        lax.fori_loop(0, B, batch_body, 0)

        bsem = pltpu.get_barrier_semaphore()
        for dz in (1, 2, 3):
            pl.semaphore_signal(
                bsem, inc=1,
                device_id=(my_x, my_y, lax.rem(mz + dz, ZDIM)),
                device_id_type=pl.DeviceIdType.MESH,
            )
        pl.semaphore_wait(bsem, 3)

        sends = []
        for dz in (1, 2, 3):
            rdma = pltpu.make_async_remote_copy(
                src_ref=comm_ref.at[0],
                dst_ref=comm_ref.at[ZDIM - dz],
                send_sem=send_sems.at[dz],
                recv_sem=recv_sems.at[ZDIM - dz],
                device_id=(my_x, my_y, lax.rem(mz + dz, ZDIM)),
                device_id_type=pl.DeviceIdType.MESH,
            )
            rdma.start()
            sends.append(rdma)

        for s in (1, 2, 3):
            recv = pltpu.make_async_remote_copy(
                src_ref=comm_ref.at[s],
                dst_ref=comm_ref.at[s],
                send_sem=send_sems.at[0],
                recv_sem=recv_sems.at[s],
                device_id=(my_x, my_y, mz),
                device_id_type=pl.DeviceIdType.MESH,
            )
            recv.wait_recv()
        for rdma in sends:
            rdma.wait_send()

        mall = comm_ref[:, 1]
        mmax = jnp.max(mall, axis=0)
        w = jnp.exp(mall - mmax[None])
        o = jnp.sum(comm_ref[:, 0] * w, axis=0)
        lsum = jnp.sum(comm_ref[:, 2] * w, axis=0)
        out_ref[:, 0] = o / lsum

    return pl.pallas_call(
        body,
        out_shape=jax.ShapeDtypeStruct((B, 1, H, D), jnp.float32),
        in_specs=[
            pl.BlockSpec(memory_space=pltpu.VMEM),
            pl.BlockSpec(memory_space=pl.ANY),
            pl.BlockSpec(memory_space=pl.ANY),
            pl.BlockSpec(memory_space=pltpu.SMEM),
            pl.BlockSpec(memory_space=pltpu.SMEM),
        ],
        out_specs=pl.BlockSpec(memory_space=pltpu.VMEM),
        scratch_shapes=[
            pltpu.VMEM((ZDIM, 3, B, H, D), jnp.float32),
            pltpu.VMEM((2, BS, H, D), jnp.float32),
            pltpu.VMEM((2, BS, H, D), jnp.float32),
            pltpu.SemaphoreType.DMA((2, 2)),
            pltpu.SemaphoreType.DMA((4,)),
            pltpu.SemaphoreType.DMA((4,)),
        ],
        compiler_params=pltpu.CompilerParams(collective_id=0),
    )(Q, K, V, cpt, counts)


# device time: 491410 ns/iter; 1.8310x vs baseline; 1.8310x over previous
import jax
import jax.numpy as jnp
from jax import lax
from jax.experimental import pallas as pl
from jax.experimental.pallas import tpu as pltpu

B = 32
NB = 256
BS = 32
H = 16
D = 128
HD = H * D
ZDIM = 4
CH = 8
CK = CH * BS


def kernel(Q, K, V, bt, lens):
    n_local_pages = K.shape[0]
    my_z = lax.axis_index("z")

    pos = jnp.arange(NB, dtype=jnp.int32)[None, :]
    owned = (bt // n_local_pages == my_z) & (pos < lens[:, None])
    order = jnp.argsort(jnp.logical_not(owned), axis=1)
    cpt = jnp.take_along_axis(bt, order, axis=1) - my_z * n_local_pages
    cpt = jnp.clip(cpt, 0, n_local_pages - 1).astype(jnp.int32)
    counts = owned.sum(axis=1).astype(jnp.int32)

    K2 = K.reshape(n_local_pages, BS, HD)
    V2 = V.reshape(n_local_pages, BS, HD)
    Q2 = (Q.reshape(B, HD) * (D ** -0.5)).astype(jnp.float32)

    def body(q_ref, k_ref, v_ref, cpt_ref, cnt_ref, out_ref,
             comm_ref, kbuf, vbuf, kv_sems, send_sems, recv_sems):
        my_x = lax.axis_index("x")
        my_y = lax.axis_index("y")
        mz = lax.axis_index("z")

        col_h = lax.broadcasted_iota(jnp.int32, (H, HD), 1) // D
        row_h = lax.broadcasted_iota(jnp.int32, (H, HD), 0)
        E16 = (col_h == row_h).astype(jnp.bfloat16)
        col_h_t = lax.broadcasted_iota(jnp.int32, (HD, H), 0) // D
        row_h_t = lax.broadcasted_iota(jnp.int32, (HD, H), 1)
        E16_T = (col_h_t == row_h_t).astype(jnp.bfloat16)

        def start_chunk(i, t, slot):
            for c in range(CH):
                p = cpt_ref[i, t * CH + c]
                pltpu.make_async_copy(
                    k_ref.at[p], kbuf.at[slot, pl.ds(c * BS, BS)],
                    kv_sems.at[0, slot, c]).start()
                pltpu.make_async_copy(
                    v_ref.at[p], vbuf.at[slot, pl.ds(c * BS, BS)],
                    kv_sems.at[1, slot, c]).start()

        def wait_chunk(slot):
            for c in range(CH):
                pltpu.make_async_copy(
                    k_ref.at[0], kbuf.at[slot, pl.ds(c * BS, BS)],
                    kv_sems.at[0, slot, c]).wait()
                pltpu.make_async_copy(
                    v_ref.at[0], vbuf.at[slot, pl.ds(c * BS, BS)],
                    kv_sems.at[1, slot, c]).wait()

        def batch_body(i, carry):
            T = cnt_ref[i]
            nch = lax.div(T + CH - 1, CH)
            q_row = q_ref[pl.ds(i, 1)]

            @pl.when(nch > 0)
            def _():
                start_chunk(i, 0, 0)

            def chunk_body(t, mla):
                m, l, acc = mla
                slot = lax.rem(t, 2)

                @pl.when(t + 1 < nch)
                def _():
                    start_chunk(i, t + 1, lax.rem(t + 1, 2))

                wait_chunk(slot)
                kc = kbuf[slot]
                vc = vbuf[slot]
                m1 = (kc * q_row).astype(jnp.bfloat16)
                s = jax.lax.dot_general(
                    m1, E16_T, (((1,), (0,)), ((), ())),
                    preferred_element_type=jnp.float32)
                entry = t * CH + lax.broadcasted_iota(jnp.int32, (CK, H), 0) // BS
                s = jnp.where(entry < T, s, -1e30)
                m_new = jnp.maximum(m, jnp.max(s, axis=0, keepdims=True))
                alpha = jnp.exp(m - m_new)
                p = jnp.exp(s - m_new).astype(jnp.bfloat16)
                l_new = l * alpha + jnp.sum(
                    jnp.exp(s - m_new), axis=0, keepdims=True)
                w = jax.lax.dot_general(
                    p, E16, (((1,), (0,)), ((), ())),
                    preferred_element_type=jnp.float32)
                alpha_flat = jax.lax.dot_general(
                    alpha.astype(jnp.bfloat16), E16, (((1,), (0,)), ((), ())),
                    preferred_element_type=jnp.float32)
                pv = jnp.sum(w * vc, axis=0, keepdims=True)
                acc_new = acc * alpha_flat + pv
                return m_new, l_new, acc_new

            init = (jnp.full((1, H), -1e30, jnp.float32),
                    jnp.zeros((1, H), jnp.float32),
                    jnp.zeros((1, HD), jnp.float32))
            m, l, acc = lax.fori_loop(0, nch, chunk_body, init)

            comm_ref[0, 0, pl.ds(i, 1)] = acc
            comm_ref[0, 1, pl.ds(i, 1)] = jnp.broadcast_to(
                m.reshape(H, 1), (H, D)).reshape(1, HD)
            comm_ref[0, 2, pl.ds(i, 1)] = jnp.broadcast_to(
                l.reshape(H, 1), (H, D)).reshape(1, HD)
            return carry

        lax.fori_loop(0, B, batch_body, 0)

        bsem = pltpu.get_barrier_semaphore()
        for dz in (1, 2, 3):
            pl.semaphore_signal(
                bsem, inc=1,
                device_id=(my_x, my_y, lax.rem(mz + dz, ZDIM)),
                device_id_type=pl.DeviceIdType.MESH,
            )
        pl.semaphore_wait(bsem, 3)

        sends = []
        for dz in (1, 2, 3):
            rdma = pltpu.make_async_remote_copy(
                src_ref=comm_ref.at[0],
                dst_ref=comm_ref.at[ZDIM - dz],
                send_sem=send_sems.at[dz],
                recv_sem=recv_sems.at[ZDIM - dz],
                device_id=(my_x, my_y, lax.rem(mz + dz, ZDIM)),
                device_id_type=pl.DeviceIdType.MESH,
            )
            rdma.start()
            sends.append(rdma)

        for s in (1, 2, 3):
            recv = pltpu.make_async_remote_copy(
                src_ref=comm_ref.at[s],
                dst_ref=comm_ref.at[s],
                send_sem=send_sems.at[0],
                recv_sem=recv_sems.at[s],
                device_id=(my_x, my_y, mz),
                device_id_type=pl.DeviceIdType.MESH,
            )
            recv.wait_recv()
        for rdma in sends:
            rdma.wait_send()

        mall = comm_ref[:, 1]
        mmax = jnp.max(mall, axis=0)
        w = jnp.exp(mall - mmax[None])
        o = jnp.sum(comm_ref[:, 0] * w, axis=0)
        lsum = jnp.sum(comm_ref[:, 2] * w, axis=0)
        out_ref[...] = o / lsum

    out_flat = pl.pallas_call(
        body,
        out_shape=jax.ShapeDtypeStruct((B, HD), jnp.float32),
        in_specs=[
            pl.BlockSpec(memory_space=pltpu.VMEM),
            pl.BlockSpec(memory_space=pl.ANY),
            pl.BlockSpec(memory_space=pl.ANY),
            pl.BlockSpec(memory_space=pltpu.SMEM),
            pl.BlockSpec(memory_space=pltpu.SMEM),
        ],
        out_specs=pl.BlockSpec(memory_space=pltpu.VMEM),
        scratch_shapes=[
            pltpu.VMEM((ZDIM, 3, B, HD), jnp.float32),
            pltpu.VMEM((2, CK, HD), jnp.float32),
            pltpu.VMEM((2, CK, HD), jnp.float32),
            pltpu.SemaphoreType.DMA((2, 2, CH)),
            pltpu.SemaphoreType.DMA((4,)),
            pltpu.SemaphoreType.DMA((4,)),
        ],
        compiler_params=pltpu.CompilerParams(collective_id=0),
    )(Q2, K2, V2, cpt, counts)
    return out_flat.reshape(B, 1, H, D)


# device time: 469910 ns/iter; 1.9147x vs baseline; 1.0458x over previous
import jax
import jax.numpy as jnp
from jax import lax
from jax.experimental import pallas as pl
from jax.experimental.pallas import tpu as pltpu

B = 32
NB = 256
BS = 32
H = 16
D = 128
HD = H * D
ZDIM = 4
CH = 8
CK = CH * BS


def kernel(Q, K, V, bt, lens):
    n_local_pages = K.shape[0]
    my_z = lax.axis_index("z")

    pos = jnp.arange(NB, dtype=jnp.int32)[None, :]
    owned = (bt // n_local_pages == my_z) & (pos < lens[:, None])
    order = jnp.argsort(jnp.logical_not(owned), axis=1)
    cpt = jnp.take_along_axis(bt, order, axis=1) - my_z * n_local_pages
    cpt = jnp.clip(cpt, 0, n_local_pages - 1).astype(jnp.int32)
    counts = owned.sum(axis=1).astype(jnp.int32)

    K2 = K.reshape(n_local_pages, BS, HD).astype(jnp.bfloat16)
    V2 = V.reshape(n_local_pages, BS, HD).astype(jnp.bfloat16)
    Q2 = (Q.reshape(B, HD) * (D ** -0.5)).astype(jnp.float32)

    def body(q_ref, k_ref, v_ref, cpt_ref, cnt_ref, out_ref,
             comm_ref, kbuf, vbuf, kv_sems, send_sems, recv_sems):
        my_x = lax.axis_index("x")
        my_y = lax.axis_index("y")
        mz = lax.axis_index("z")

        col_h = lax.broadcasted_iota(jnp.int32, (H, HD), 1) // D
        row_h = lax.broadcasted_iota(jnp.int32, (H, HD), 0)
        E16 = (col_h == row_h).astype(jnp.bfloat16)
        col_h_t = lax.broadcasted_iota(jnp.int32, (HD, H), 0) // D
        row_h_t = lax.broadcasted_iota(jnp.int32, (HD, H), 1)
        E16_T = (col_h_t == row_h_t).astype(jnp.bfloat16)

        def start_chunk(i, t, slot):
            for c in range(CH):
                p = cpt_ref[i, t * CH + c]
                pltpu.make_async_copy(
                    k_ref.at[p], kbuf.at[slot, pl.ds(c * BS, BS)],
                    kv_sems.at[0, slot, c]).start()
                pltpu.make_async_copy(
                    v_ref.at[p], vbuf.at[slot, pl.ds(c * BS, BS)],
                    kv_sems.at[1, slot, c]).start()

        def wait_chunk(slot):
            for c in range(CH):
                pltpu.make_async_copy(
                    k_ref.at[0], kbuf.at[slot, pl.ds(c * BS, BS)],
                    kv_sems.at[0, slot, c]).wait()
                pltpu.make_async_copy(
                    v_ref.at[0], vbuf.at[slot, pl.ds(c * BS, BS)],
                    kv_sems.at[1, slot, c]).wait()

        def batch_body(i, carry):
            T = cnt_ref[i]
            nch = lax.div(T + CH - 1, CH)
            q_row = q_ref[pl.ds(i, 1)].astype(jnp.bfloat16)

            @pl.when(nch > 0)
            def _():
                start_chunk(i, 0, 0)

            def chunk_body(t, mla):
                m, l, acc = mla
                slot = lax.rem(t, 2)

                @pl.when(t + 1 < nch)
                def _():
                    start_chunk(i, t + 1, lax.rem(t + 1, 2))

                wait_chunk(slot)
                kc = kbuf[slot]
                vc = vbuf[slot]
                m1 = kc * q_row
                s = jax.lax.dot_general(
                    m1, E16_T, (((1,), (0,)), ((), ())),
                    preferred_element_type=jnp.float32)
                entry = t * CH + lax.broadcasted_iota(jnp.int32, (CK, H), 0) // BS
                s = jnp.where(entry < T, s, -1e30)
                m_new = jnp.maximum(m, jnp.max(s, axis=0, keepdims=True))
                alpha = jnp.exp(m - m_new)
                p = jnp.exp(s - m_new).astype(jnp.bfloat16)
                l_new = l * alpha + jnp.sum(
                    jnp.exp(s - m_new), axis=0, keepdims=True)
                w = jax.lax.dot_general(
                    p, E16, (((1,), (0,)), ((), ())),
                    preferred_element_type=jnp.float32)
                alpha_flat = jax.lax.dot_general(
                    alpha.astype(jnp.bfloat16), E16, (((1,), (0,)), ((), ())),
                    preferred_element_type=jnp.float32)
                pv = jnp.sum(w * vc.astype(jnp.float32),
                             axis=0, keepdims=True)
                acc_new = acc * alpha_flat + pv
                return m_new, l_new, acc_new

            init = (jnp.full((1, H), -1e30, jnp.float32),
                    jnp.zeros((1, H), jnp.float32),
                    jnp.zeros((1, HD), jnp.float32))
            m, l, acc = lax.fori_loop(0, nch, chunk_body, init)

            comm_ref[0, 0, pl.ds(i, 1)] = acc
            comm_ref[0, 1, pl.ds(i, 1)] = jnp.broadcast_to(
                m.reshape(H, 1), (H, D)).reshape(1, HD)
            comm_ref[0, 2, pl.ds(i, 1)] = jnp.broadcast_to(
                l.reshape(H, 1), (H, D)).reshape(1, HD)
            return carry

        lax.fori_loop(0, B, batch_body, 0)

        bsem = pltpu.get_barrier_semaphore()
        for dz in (1, 2, 3):
            pl.semaphore_signal(
                bsem, inc=1,
                device_id=(my_x, my_y, lax.rem(mz + dz, ZDIM)),
                device_id_type=pl.DeviceIdType.MESH,
            )
        pl.semaphore_wait(bsem, 3)

        sends = []
        for dz in (1, 2, 3):
            rdma = pltpu.make_async_remote_copy(
                src_ref=comm_ref.at[0],
                dst_ref=comm_ref.at[ZDIM - dz],
                send_sem=send_sems.at[dz],
                recv_sem=recv_sems.at[ZDIM - dz],
                device_id=(my_x, my_y, lax.rem(mz + dz, ZDIM)),
                device_id_type=pl.DeviceIdType.MESH,
            )
            rdma.start()
            sends.append(rdma)

        for s in (1, 2, 3):
            recv = pltpu.make_async_remote_copy(
                src_ref=comm_ref.at[s],
                dst_ref=comm_ref.at[s],
                send_sem=send_sems.at[0],
                recv_sem=recv_sems.at[s],
                device_id=(my_x, my_y, mz),
                device_id_type=pl.DeviceIdType.MESH,
            )
            recv.wait_recv()
        for rdma in sends:
            rdma.wait_send()

        mall = comm_ref[:, 1]
        mmax = jnp.max(mall, axis=0)
        w = jnp.exp(mall - mmax[None])
        o = jnp.sum(comm_ref[:, 0] * w, axis=0)
        lsum = jnp.sum(comm_ref[:, 2] * w, axis=0)
        out_ref[...] = o / lsum

    out_flat = pl.pallas_call(
        body,
        out_shape=jax.ShapeDtypeStruct((B, HD), jnp.float32),
        in_specs=[
            pl.BlockSpec(memory_space=pltpu.VMEM),
            pl.BlockSpec(memory_space=pl.ANY),
            pl.BlockSpec(memory_space=pl.ANY),
            pl.BlockSpec(memory_space=pltpu.SMEM),
            pl.BlockSpec(memory_space=pltpu.SMEM),
        ],
        out_specs=pl.BlockSpec(memory_space=pltpu.VMEM),
        scratch_shapes=[
            pltpu.VMEM((ZDIM, 3, B, HD), jnp.float32),
            pltpu.VMEM((2, CK, HD), jnp.bfloat16),
            pltpu.VMEM((2, CK, HD), jnp.bfloat16),
            pltpu.SemaphoreType.DMA((2, 2, CH)),
            pltpu.SemaphoreType.DMA((4,)),
            pltpu.SemaphoreType.DMA((4,)),
        ],
        compiler_params=pltpu.CompilerParams(collective_id=0),
    )(Q2, K2, V2, cpt, counts)
    return out_flat.reshape(B, 1, H, D)


# device time: 407308 ns/iter; 2.2090x vs baseline; 1.1537x over previous
import jax
import jax.numpy as jnp
from jax import lax
from jax.experimental import pallas as pl
from jax.experimental.pallas import tpu as pltpu

B = 32
NB = 256
BS = 32
H = 16
D = 128
ZDIM = 4
CH = 8
CK = CH * BS


def kernel(Q, K, V, bt, lens):
    n_local_pages = K.shape[0]
    my_z = lax.axis_index("z")

    pos = jnp.arange(NB, dtype=jnp.int32)[None, :]
    owned = (bt // n_local_pages == my_z) & (pos < lens[:, None])
    order = jnp.argsort(jnp.logical_not(owned), axis=1)
    cpt = jnp.take_along_axis(bt, order, axis=1) - my_z * n_local_pages
    cpt = jnp.clip(cpt, 0, n_local_pages - 1).astype(jnp.int32)
    counts = owned.sum(axis=1).astype(jnp.int32)

    scale = D ** -0.5

    def body(q_ref, k_ref, v_ref, cpt_ref, cnt_ref, out_ref,
             comm_ref, kbuf, vbuf, kv_sems, send_sems, recv_sems):
        my_x = lax.axis_index("x")
        my_y = lax.axis_index("y")
        mz = lax.axis_index("z")

        ones_col = jnp.ones((D, 1), jnp.bfloat16)

        def start_chunk(i, t, slot):
            for c in range(CH):
                p = cpt_ref[i, t * CH + c]
                pltpu.make_async_copy(
                    k_ref.at[p], kbuf.at[slot, c],
                    kv_sems.at[0, slot, c]).start()
                pltpu.make_async_copy(
                    v_ref.at[p], vbuf.at[slot, c],
                    kv_sems.at[1, slot, c]).start()

        def wait_chunk(slot):
            for c in range(CH):
                pltpu.make_async_copy(
                    k_ref.at[0], kbuf.at[slot, c],
                    kv_sems.at[0, slot, c]).wait()
                pltpu.make_async_copy(
                    v_ref.at[0], vbuf.at[slot, c],
                    kv_sems.at[1, slot, c]).wait()

        def batch_body(i, carry):
            T = cnt_ref[i]
            nch = lax.div(T + CH - 1, CH)
            qb = q_ref[i, 0] * scale

            @pl.when(nch > 0)
            def _():
                start_chunk(i, 0, 0)

            def chunk_body(t, mla):
                m, l, acc = mla
                slot = lax.rem(t, 2)

                @pl.when(t + 1 < nch)
                def _():
                    start_chunk(i, t + 1, lax.rem(t + 1, 2))

                wait_chunk(slot)
                kc = kbuf[slot].reshape(CK, H, D)
                vc = vbuf[slot].reshape(CK, H, D)
                m1 = (kc * qb[None]).astype(jnp.bfloat16).reshape(CK * H, D)
                s2 = jax.lax.dot_general(
                    m1, ones_col, (((1,), (0,)), ((), ())),
                    preferred_element_type=jnp.float32)
                s = s2.reshape(CK, H, 1)
                entry = t * CH + \
                    lax.broadcasted_iota(jnp.int32, (CK, H, 1), 0) // BS
                s = jnp.where(entry < T, s, -1e30)
                m_new = jnp.maximum(m, jnp.max(s, axis=0))
                alpha = jnp.exp(m - m_new)
                p = jnp.exp(s - m_new[None])
                l_new = l * alpha + jnp.sum(p, axis=0)
                pv = jnp.sum(jnp.broadcast_to(p, (CK, H, D)) * vc, axis=0)
                acc_new = acc * alpha + pv
                return m_new, l_new, acc_new

            init = (jnp.full((H, 1), -1e30, jnp.float32),
                    jnp.zeros((H, 1), jnp.float32),
                    jnp.zeros((H, D), jnp.float32))
            m, l, acc = lax.fori_loop(0, nch, chunk_body, init)

            comm_ref[0, 0, i] = acc
            comm_ref[0, 1, i] = jnp.broadcast_to(m, (H, D))
            comm_ref[0, 2, i] = jnp.broadcast_to(l, (H, D))
            return carry

        lax.fori_loop(0, B, batch_body, 0)

        bsem = pltpu.get_barrier_semaphore()
        for dz in (1, 2, 3):
            pl.semaphore_signal(
                bsem, inc=1,
                device_id=(my_x, my_y, lax.rem(mz + dz, ZDIM)),
                device_id_type=pl.DeviceIdType.MESH,
            )
        pl.semaphore_wait(bsem, 3)

        sends = []
        for dz in (1, 2, 3):
            rdma = pltpu.make_async_remote_copy(
                src_ref=comm_ref.at[0],
                dst_ref=comm_ref.at[ZDIM - dz],
                send_sem=send_sems.at[dz],
                recv_sem=recv_sems.at[ZDIM - dz],
                device_id=(my_x, my_y, lax.rem(mz + dz, ZDIM)),
                device_id_type=pl.DeviceIdType.MESH,
            )
            rdma.start()
            sends.append(rdma)

        for s in (1, 2, 3):
            recv = pltpu.make_async_remote_copy(
                src_ref=comm_ref.at[s],
                dst_ref=comm_ref.at[s],
                send_sem=send_sems.at[0],
                recv_sem=recv_sems.at[s],
                device_id=(my_x, my_y, mz),
                device_id_type=pl.DeviceIdType.MESH,
            )
            recv.wait_recv()
        for rdma in sends:
            rdma.wait_send()

        mall = comm_ref[:, 1]
        mmax = jnp.max(mall, axis=0)
        w = jnp.exp(mall - mmax[None])
        o = jnp.sum(comm_ref[:, 0] * w, axis=0)
        lsum = jnp.sum(comm_ref[:, 2] * w, axis=0)
        out_ref[:, 0] = o / lsum

    return pl.pallas_call(
        body,
        out_shape=jax.ShapeDtypeStruct((B, 1, H, D), jnp.float32),
        in_specs=[
            pl.BlockSpec(memory_space=pltpu.VMEM),
            pl.BlockSpec(memory_space=pl.ANY),
            pl.BlockSpec(memory_space=pl.ANY),
            pl.BlockSpec(memory_space=pltpu.SMEM),
            pl.BlockSpec(memory_space=pltpu.SMEM),
        ],
        out_specs=pl.BlockSpec(memory_space=pltpu.VMEM),
        scratch_shapes=[
            pltpu.VMEM((ZDIM, 3, B, H, D), jnp.float32),
            pltpu.VMEM((2, CH, BS, H, D), jnp.float32),
            pltpu.VMEM((2, CH, BS, H, D), jnp.float32),
            pltpu.SemaphoreType.DMA((2, 2, CH)),
            pltpu.SemaphoreType.DMA((4,)),
            pltpu.SemaphoreType.DMA((4,)),
        ],
        compiler_params=pltpu.CompilerParams(collective_id=0),
    )(Q, K, V, cpt, counts)
